# baseline (speedup 1.0000x reference)
import jax
import jax.numpy as jnp
from jax.experimental import pallas as pl
from jax.experimental.pallas import tpu as pltpu

_BL = 256


def _body(xlen_ref, x_ref, o_ref):
    b = pl.program_id(0)
    j = pl.program_id(1)
    l = xlen_ref[b]
    row = jax.lax.broadcasted_iota(jnp.int32, o_ref.shape, 1) + j * _BL
    o_ref[...] = jnp.where(row < l, x_ref[...], 0.0)


def kernel(x, x_len):
    B, L, D = x.shape
    xlen = x_len.astype(jnp.int32)
    nj = L // _BL

    def x_map(b, j, xlen_ref):
        last = jnp.maximum((xlen_ref[b] + _BL - 1) // _BL - 1, 0)
        return (b, jnp.minimum(j, last), 0)

    grid_spec = pltpu.PrefetchScalarGridSpec(
        num_scalar_prefetch=1,
        grid=(B, nj),
        in_specs=[pl.BlockSpec((1, _BL, D), x_map)],
        out_specs=pl.BlockSpec((1, _BL, D), lambda b, j, xlen_ref: (b, j, 0)),
    )
    return pl.pallas_call(
        _body,
        grid_spec=grid_spec,
        out_shape=jax.ShapeDtypeStruct((B, L, D), x.dtype),
    )(xlen, x)


# speedup vs baseline: 1.2590x; 1.2590x over previous
import jax
import jax.numpy as jnp
from jax.experimental import pallas as pl
from jax.experimental.pallas import tpu as pltpu

_BL = 512


def _body(xlen_ref, x_ref, o_ref):
    b = pl.program_id(0)
    j = pl.program_id(1)
    l = xlen_ref[b]
    row = jax.lax.broadcasted_iota(jnp.int32, o_ref.shape, 1) + j * _BL
    o_ref[...] = jnp.where(row < l, x_ref[...], 0.0)


def kernel(x, x_len):
    B, L, D = x.shape
    xlen = x_len.astype(jnp.int32)
    nj = L // _BL

    def x_map(b, j, xlen_ref):
        last = jnp.maximum((xlen_ref[b] + _BL - 1) // _BL - 1, 0)
        return (b, jnp.minimum(j, last), 0)

    grid_spec = pltpu.PrefetchScalarGridSpec(
        num_scalar_prefetch=1,
        grid=(B, nj),
        in_specs=[pl.BlockSpec((1, _BL, D), x_map)],
        out_specs=pl.BlockSpec((1, _BL, D), lambda b, j, xlen_ref: (b, j, 0)),
    )
    return pl.pallas_call(
        _body,
        grid_spec=grid_spec,
        out_shape=jax.ShapeDtypeStruct((B, L, D), x.dtype),
    )(xlen, x)
